# baseline (speedup 1.0000x reference)
import functools

import jax
import jax.numpy as jnp
import numpy as np
from jax import lax
from jax.experimental import pallas as pl
from jax.experimental.pallas import tpu as pltpu
from jax.experimental.pallas import tpu_sc as plsc

VOCAB = 1000000
DIM = 64
B = 4096
L = 200
N_SEG = 3

_NC = 2
_NS = 16
_NW = _NC * _NS
_N = B * L
_PW = _N // _NW
_SUB = 128
_NSUB = 4
_CH = _SUB * _NSUB
_NCHUNK = _PW // _CH

_FB = 2048


def _sinusoidal_pe(length, dim):
    pos = np.arange(length)[:, None].astype(np.float64)
    i = np.arange(dim)[None, :]
    angle_rates = 1.0 / np.power(10000.0, (2 * (i // 2)) / np.float64(dim))
    angles = pos * angle_rates
    pe = np.zeros((length, dim), dtype=np.float64)
    pe[:, 0::2] = np.sin(angles[:, 0::2])
    pe[:, 1::2] = np.cos(angles[:, 1::2])
    return pe.astype(np.float32)


_PE = _sinusoidal_pe(L, DIM)


def _combo_table(seg_table):
    def body(seg_ref, pe_ref, out_ref):
        out_ref[...] = seg_ref[...] + pe_ref[...]

    out = pl.pallas_call(
        body,
        out_shape=jax.ShapeDtypeStruct((N_SEG, L, DIM), jnp.float32),
    )(seg_table[:, None, :], jnp.asarray(_PE)[None, :, :])
    return out.reshape(N_SEG * L, DIM)


def _sc_lookup(seq_t, lab_t, token_table, combo):
    mesh = plsc.VectorSubcoreMesh(core_axis_name="c", subcore_axis_name="s")

    @functools.partial(
        pl.kernel,
        out_type=jax.ShapeDtypeStruct((_N, DIM), jnp.float32),
        mesh=mesh,
        compiler_params=pltpu.CompilerParams(use_tc_tiling_on_sc=False,
                                             needs_layout_passes=False),
        scratch_types=[
            pltpu.VMEM((2, _CH), jnp.int32),
            pltpu.VMEM((2, _CH), jnp.int32),
            pltpu.VMEM((2, _NSUB, _SUB), jnp.int32),
            pltpu.VMEM((2, _NSUB, _SUB), jnp.int32),
            pltpu.VMEM((2, _CH, DIM), jnp.float32),
            pltpu.SemaphoreType.DMA,
            pltpu.SemaphoreType.DMA,
            pltpu.SemaphoreType.DMA,
        ],
    )
    def k(seq_hbm, lab_hbm, tok_hbm, combo_hbm, out_hbm,
          idx_v, lab_v, til_v, cil_v, rows_v, sem_c, sem_t, sem_o):
        wid = lax.axis_index("s") * _NC + lax.axis_index("c")
        lane = lax.iota(jnp.int32, 16)
        gsub = (lane & 1) * (_CH // 2) + (lane >> 1)

        def stage(kk):
            f0 = wid * _PW + kk * _CH
            l = f0 // B
            r0 = f0 - l * B
            q0 = r0 % 2048
            ba = (r0 // 2048) * 2048 + q0 // 2
            return pl.multiple_of(f0, _CH), l, pl.multiple_of(ba, _CH // 2)

        def load_idx(kk, buf):
            _, l, ba = stage(kk)
            half = _CH // 2
            pltpu.sync_copy(seq_hbm.at[l, pl.ds(ba, half)],
                            idx_v.at[buf, pl.ds(0, half)])
            pltpu.sync_copy(seq_hbm.at[l, pl.ds(ba + 1024, half)],
                            idx_v.at[buf, pl.ds(half, half)])
            pltpu.sync_copy(lab_hbm.at[l, pl.ds(ba, half)],
                            lab_v.at[buf, pl.ds(0, half)])
            pltpu.sync_copy(lab_hbm.at[l, pl.ds(ba + 1024, half)],
                            lab_v.at[buf, pl.ds(half, half)])
            bufv = lane * 0 + buf
            for c in range(_CH // 16):
                gi = c * 8 + gsub
                til_v[buf, c // 8, pl.ds((c % 8) * 16, 16)] = plsc.load_gather(
                    idx_v, [bufv, gi])
                cil_v[buf, c // 8, pl.ds((c % 8) * 16, 16)] = plsc.load_gather(
                    lab_v, [bufv, gi]) * L + l

        def combo_cps(kk, buf):
            return [pltpu.async_copy(
                        combo_hbm.at[cil_v.at[buf, j]],
                        rows_v.at[buf, pl.ds(j * _SUB, _SUB)], sem_c)
                    for j in range(_NSUB)]

        def token_cps(kk, buf):
            return [pltpu.async_copy(
                        tok_hbm.at[til_v.at[buf, j]],
                        rows_v.at[buf, pl.ds(j * _SUB, _SUB)], sem_t, add=True)
                    for j in range(_NSUB)]

        def out_dst(kk):
            f0, _, _ = stage(kk)
            return out_hbm.at[pl.ds(f0, _CH)]

        def out_cp(kk, buf):
            pltpu.async_copy(rows_v.at[buf], out_dst(kk), sem_o)

        def out_wait(kk, buf):
            pltpu.make_async_copy(rows_v.at[buf], out_dst(kk), sem_o).wait()

        def chunk_body(kk, carry):
            buf = kk % 2
            load_idx(kk, buf)
            for cp in combo_cps(kk, buf):
                cp.wait()
            for cp in token_cps(kk, buf):
                cp.wait()
            out_cp(kk, buf)
            @pl.when(kk >= 1)
            def _drain_prev_write():
                out_wait(kk - 1, (kk - 1) % 2)
            return carry

        lax.fori_loop(0, _NCHUNK, chunk_body, 0)
        out_wait(_NCHUNK - 1, (_NCHUNK - 1) % 2)

    return k(seq_t, lab_t, token_table, combo)


def _finisher(pairs):
    def body(in_ref, out_ref):
        y = in_ref[...].T
        out_ref[0] = jnp.concatenate([y[:DIM], y[DIM:]], axis=1)

    nb = B // _FB
    return pl.pallas_call(
        body,
        grid=(L, nb),
        in_specs=[pl.BlockSpec((_FB // 2, 2 * DIM), lambda l, c: (l * nb + c, 0))],
        out_specs=pl.BlockSpec((1, DIM, _FB), lambda l, c: (l, 0, c)),
        out_shape=jax.ShapeDtypeStruct((L, DIM, B), jnp.float32),
    )(pairs)


def kernel(seq, seg_label, token_table, seg_table):
    combo = _combo_table(seg_table)
    out2d = _sc_lookup(seq.T, seg_label.T, token_table, combo)
    out_t = _finisher(out2d.reshape(_N * DIM // 128, 128))
    return out_t.transpose(2, 0, 1)


# speedup vs baseline: 1.2156x; 1.2156x over previous
import functools

import jax
import jax.numpy as jnp
import numpy as np
from jax import lax
from jax.experimental import pallas as pl
from jax.experimental.pallas import tpu as pltpu
from jax.experimental.pallas import tpu_sc as plsc

VOCAB = 1000000
DIM = 64
B = 4096
L = 200
N_SEG = 3

_NC = 2
_NS = 16
_NW = _NC * _NS
_N = B * L
_PW = _N // _NW
_SUB = 128
_NSUB = 4
_CH = _SUB * _NSUB
_NCHUNK = _PW // _CH


def _sinusoidal_pe(length, dim):
    pos = np.arange(length)[:, None].astype(np.float64)
    i = np.arange(dim)[None, :]
    angle_rates = 1.0 / np.power(10000.0, (2 * (i // 2)) / np.float64(dim))
    angles = pos * angle_rates
    pe = np.zeros((length, dim), dtype=np.float64)
    pe[:, 0::2] = np.sin(angles[:, 0::2])
    pe[:, 1::2] = np.cos(angles[:, 1::2])
    return pe.astype(np.float32)


_PE = _sinusoidal_pe(L, DIM)


def _combo_table(seg_table):
    def body(seg_ref, pe_ref, out_ref):
        out_ref[...] = seg_ref[...] + pe_ref[...]

    out = pl.pallas_call(
        body,
        out_shape=jax.ShapeDtypeStruct((N_SEG, L, DIM), jnp.float32),
    )(seg_table[:, None, :], jnp.asarray(_PE)[None, :, :])
    return out.reshape(N_SEG * L, DIM)


def _sc_lookup(seq2d, lab2d, token_table, combo):
    mesh = plsc.VectorSubcoreMesh(core_axis_name="c", subcore_axis_name="s")

    @functools.partial(
        pl.kernel,
        out_type=jax.ShapeDtypeStruct((_N, DIM), jnp.float32),
        mesh=mesh,
        compiler_params=pltpu.CompilerParams(use_tc_tiling_on_sc=False),
        scratch_types=[
            pltpu.VMEM((_NSUB, _SUB), jnp.int32),
            pltpu.VMEM((_NSUB, _SUB), jnp.int32),
            pltpu.VMEM((_NSUB, _SUB), jnp.int32),
            pltpu.VMEM((_NSUB, _SUB, DIM), jnp.float32),
            pltpu.SemaphoreType.DMA,
            pltpu.SemaphoreType.DMA,
        ],
    )
    def k(seq_hbm, lab_hbm, tok_hbm, combo_hbm, out_hbm,
          idx_v, lab_v, ci_v, rows_t, sem_t, sem_a):
        wid = lax.axis_index("s") * _NC + lax.axis_index("c")
        lane = lax.iota(jnp.int32, 16)

        def chunk_body(kk, carry):
            r0 = wid * (_PW // _SUB) + kk * _NSUB
            pltpu.sync_copy(seq_hbm.at[pl.ds(r0, _NSUB)], idx_v)
            pltpu.sync_copy(lab_hbm.at[pl.ds(r0, _NSUB)], lab_v)
            base = kk * _CH
            for j in range(_NSUB):
                for c in range(_SUB // 16):
                    pos = lax.rem(base + j * _SUB + c * 16 + lane, L)
                    ci_v[j, pl.ds(c * 16, 16)] = lab_v[j, pl.ds(c * 16, 16)] * L + pos
            cps = []
            for j in range(_NSUB):
                cps.append(pltpu.async_copy(combo_hbm.at[ci_v.at[j]], rows_t.at[j], sem_a))
            for cp in cps:
                cp.wait()
            cps = []
            for j in range(_NSUB):
                cps.append(pltpu.async_copy(tok_hbm.at[idx_v.at[j]], rows_t.at[j], sem_t, add=True))
            for cp in cps:
                cp.wait()
            row_out = wid * _PW + kk * _CH
            for j in range(_NSUB):
                pltpu.sync_copy(rows_t.at[j], out_hbm.at[pl.ds(row_out + j * _SUB, _SUB)])
            return carry

        lax.fori_loop(0, _NCHUNK, chunk_body, 0)

    return k(seq2d, lab2d, token_table, combo)


def kernel(seq, seg_label, token_table, seg_table):
    combo = _combo_table(seg_table)
    seq2d = seq.reshape(_N // _SUB, _SUB).astype(jnp.int32)
    lab2d = seg_label.reshape(_N // _SUB, _SUB).astype(jnp.int32)
    out = _sc_lookup(seq2d, lab2d, token_table, combo)
    return out.reshape(B, L, DIM)
